# baseline (speedup 1.0000x reference)
import functools

import jax
import jax.numpy as jnp
from jax import lax
from jax.experimental import pallas as pl
from jax.experimental.pallas import tpu as pltpu
from jax.experimental.pallas import tpu_sc as plsc

_JC = 8


def _make_sc_call(B, H, W, D):
    mesh = plsc.VectorSubcoreMesh(core_axis_name="c", subcore_axis_name="s")
    NC = 2

    @functools.partial(
        pl.kernel,
        mesh=mesh,
        out_type=jax.ShapeDtypeStruct((B, H, H, W, D), jnp.float32),
        scratch_types=[
            pltpu.VMEM((2 * H, D), jnp.float32),
            pltpu.VMEM((2 * W, D), jnp.float32),
            pltpu.VMEM((_JC, W, D), jnp.float32),
        ],
    )
    def sc_call(whr_hbm, wwr_hbm, out_hbm, whr_v, wwr_v, rows_v):
        wid = lax.axis_index("s") * NC + lax.axis_index("c")
        pltpu.sync_copy(whr_hbm, whr_v)
        pltpu.sync_copy(wwr_hbm, wwr_v)

        for jc in range(H // _JC):
            def j_body(jl, _):
                j = jc * _JC + jl
                r1 = (H - 1) - wid + j
                wh = [whr_v[r1, pl.ds(v * 16, 16)] for v in range(D // 16)]
                for k in range(W):
                    r2 = (W - 1) - j + k
                    for v in range(D // 16):
                        rows_v[jl, k, pl.ds(v * 16, 16)] = (
                            wh[v] + wwr_v[r2, pl.ds(v * 16, 16)]
                        )
                return ()

            lax.fori_loop(0, _JC, j_body, ())
            for b in range(B):
                pltpu.sync_copy(
                    rows_v, out_hbm.at[b, wid, pl.ds(jc * _JC, _JC)]
                )

    return sc_call


def kernel(x, Wh, Ww):
    B, C, H, W = x.shape
    D = Wh.shape[1]
    Whr = jnp.concatenate([Wh[::-1], jnp.zeros((1, D), Wh.dtype)], axis=0)
    Wwr = jnp.concatenate([Ww[::-1], jnp.zeros((1, D), Ww.dtype)], axis=0)
    return _make_sc_call(B, H, W, D)(Whr, Wwr)


# speedup vs baseline: 2.4028x; 2.4028x over previous
import jax
import jax.numpy as jnp
from jax.experimental import pallas as pl
from jax.experimental.pallas import tpu as pltpu


def _body(tab_ref, out_ref, eh_ref, ew_ref):
    b = pl.program_id(0)
    ib = pl.program_id(1)
    _, BI, H, W, D = out_ref.shape

    @pl.when(jnp.logical_and(b == 0, ib == 0))
    def _init():
        for i in range(H):
            eh_ref[i] = tab_ref[pl.ds(H - 1 - i, H), :]
        for j in range(W):
            ew_ref[j] = tab_ref[pl.ds(3 * W - 2 - j, W), :]

    eh = eh_ref[pl.ds(ib * BI, BI)]
    ew = ew_ref[...]
    out_ref[0] = eh[:, :, None, :] + ew[None, :, :, :]


def kernel(x, Wh, Ww):
    B, C, H, W = x.shape
    D = Wh.shape[1]
    BI = 8
    tab = jnp.concatenate([Ww, Wh], axis=0)[::-1]
    return pl.pallas_call(
        _body,
        grid=(B, H // BI),
        in_specs=[
            pl.BlockSpec((2 * (H + W) - 2, D), lambda b, ib: (0, 0)),
        ],
        out_specs=pl.BlockSpec((1, BI, H, W, D), lambda b, ib: (b, ib, 0, 0, 0)),
        out_shape=jax.ShapeDtypeStruct((B, H, H, W, D), jnp.float32),
        scratch_shapes=[
            pltpu.VMEM((H, H, D), jnp.float32),
            pltpu.VMEM((W, W, D), jnp.float32),
        ],
    )(tab)
